# baseline (speedup 1.0000x reference)
import functools

import jax
import jax.numpy as jnp
from jax import lax
from jax.experimental import pallas as pl
from jax.experimental.pallas import tpu as pltpu
from jax.experimental.pallas import tpu_sc as plsc

D_MODEL = 768
N_EXP = 64
TEMP = 0.1
N_TOK = 32768
BT = 4096
NCHUNK = 1
CH = N_TOK // NCHUNK

_info = plsc.get_sparse_core_info()
_NC, _NS, _L = _info.num_cores, _info.num_subcores, _info.num_lanes
_NW = _NC * _NS


def _logits_body(x_ref, wt_ref, b_ref, logits_ref, logits_t_ref):
    logits = (
        jnp.dot(x_ref[...], wt_ref[...], preferred_element_type=jnp.float32)
        + b_ref[...][None, :]) / TEMP
    logits_ref[...] = logits
    logits_t_ref[...] = logits.T


def _logits_body_acc(x_ref, wt_ref, b_ref, prev_ref, logits_ref, logits_t_ref):
    del prev_ref
    _logits_body(x_ref, wt_ref, b_ref, logits_ref, logits_t_ref)


def _make_sc_topk(n_tok):
    tok_w = n_tok // _NW
    grp = tok_w // _L
    unroll = min(4, grp)
    mesh = plsc.VectorSubcoreMesh(core_axis_name="c", subcore_axis_name="s")

    npiece = 4 if tok_w % (4 * unroll * _L) == 0 else 1

    @functools.partial(
        pl.kernel,
        mesh=mesh,
        out_type=[
            jax.ShapeDtypeStruct((2, n_tok), jnp.float32),
            jax.ShapeDtypeStruct((2, n_tok), jnp.int32),
        ],
        scratch_types=[
            pltpu.VMEM((N_EXP, tok_w), jnp.float32),
            pltpu.VMEM((tok_w,), jnp.float32),
            pltpu.VMEM((tok_w,), jnp.float32),
            pltpu.VMEM((tok_w,), jnp.int32),
            pltpu.VMEM((tok_w,), jnp.int32),
        ] + [pltpu.SemaphoreType.DMA] * npiece,
    )
    def _sc_topk(logits_t_hbm, probs_hbm, idx_hbm, lt_v, p1_v, p2_v, i1_v, i2_v,
                 *sems):
        wid = lax.axis_index("s") * _NC + lax.axis_index("c")
        base = wid * tok_w
        piece = tok_w // npiece
        copies = [
            pltpu.async_copy(
                logits_t_hbm.at[:, pl.ds(base + h * piece, piece)],
                lt_v.at[:, pl.ds(h * piece, piece)],
                sems[h])
            for h in range(npiece)
        ]

        neg = jnp.full((_L,), -jnp.inf, jnp.float32)
        zero = jnp.zeros((_L,), jnp.int32)

        def super_group(sg, _):
            offs = [sg * (unroll * _L) + g * _L for g in range(unroll)]
            m1 = [neg] * unroll
            m2 = [neg] * unroll
            j1 = [zero] * unroll
            j2 = [zero] * unroll
            for e in range(N_EXP):
                ei = jnp.full((_L,), e, jnp.int32)
                for g in range(unroll):
                    v = lt_v[e, pl.ds(offs[g], _L)]
                    gt1 = v > m1[g]
                    lose = jnp.minimum(v, m1[g])
                    gt2 = lose > m2[g]
                    nj1 = jnp.where(gt1, ei, j1[g])
                    tj = jnp.where(gt1, j1[g], ei)
                    j2[g] = jnp.where(gt2, tj, j2[g])
                    m1[g] = jnp.maximum(v, m1[g])
                    m2[g] = jnp.maximum(lose, m2[g])
                    j1[g] = nj1
            for g in range(unroll):
                e2 = jnp.exp(m2[g] - m1[g])
                p1 = 1.0 / (1.0 + e2)
                p1_v[pl.ds(offs[g], _L)] = p1
                p2_v[pl.ds(offs[g], _L)] = e2 * p1
                i1_v[pl.ds(offs[g], _L)] = j1[g]
                i2_v[pl.ds(offs[g], _L)] = j2[g]
            return 0

        spg = grp // unroll // npiece
        for h in range(npiece):
            copies[h].wait()
            lax.fori_loop(h * spg, (h + 1) * spg, super_group, 0)

        pltpu.sync_copy(p1_v, probs_hbm.at[0, pl.ds(base, tok_w)])
        pltpu.sync_copy(p2_v, probs_hbm.at[1, pl.ds(base, tok_w)])
        pltpu.sync_copy(i1_v, idx_hbm.at[0, pl.ds(base, tok_w)])
        pltpu.sync_copy(i2_v, idx_hbm.at[1, pl.ds(base, tok_w)])

    return _sc_topk


_sc_topk_chunk = _make_sc_topk(CH)


@jax.jit
def kernel(x, W, b):
    wt = W.T
    nblk = CH // BT
    logits = None
    probs_parts, idx_parts = [], []
    for c in range(NCHUNK):
        c0 = c * nblk
        x_spec = pl.BlockSpec((BT, D_MODEL), lambda i, c0=c0: (c0 + i, 0))
        w_spec = pl.BlockSpec((D_MODEL, N_EXP), lambda i: (0, 0))
        b_spec = pl.BlockSpec((N_EXP,), lambda i: (0,))
        out_specs = [
            pl.BlockSpec((BT, N_EXP), lambda i, c0=c0: (c0 + i, 0)),
            pl.BlockSpec((N_EXP, BT), lambda i: (0, i)),
        ]
        out_shape = [
            jax.ShapeDtypeStruct((N_TOK, N_EXP), jnp.float32),
            jax.ShapeDtypeStruct((N_EXP, CH), jnp.float32),
        ]
        if c == 0:
            logits, lt_c = pl.pallas_call(
                _logits_body,
                grid=(nblk,),
                in_specs=[x_spec, w_spec, b_spec],
                out_specs=out_specs,
                out_shape=out_shape,
            )(x, wt, b)
        else:
            logits, lt_c = pl.pallas_call(
                _logits_body_acc,
                grid=(nblk,),
                in_specs=[x_spec, w_spec, b_spec,
                          pl.BlockSpec(memory_space=pl.ANY)],
                out_specs=out_specs,
                out_shape=out_shape,
                input_output_aliases={3: 0},
            )(x, wt, b, logits)
        probs_c, idx_c = _sc_topk_chunk(lt_c)
        probs_parts.append(probs_c)
        idx_parts.append(idx_c)
    probs_t = jnp.concatenate(probs_parts, axis=1)
    idx_t = jnp.concatenate(idx_parts, axis=1)
    return (logits, probs_t.T, idx_t.T)


# speedup vs baseline: 1.0369x; 1.0369x over previous
import functools

import jax
import jax.numpy as jnp
from jax import lax
from jax.experimental import pallas as pl
from jax.experimental.pallas import tpu as pltpu
from jax.experimental.pallas import tpu_sc as plsc

D_MODEL = 768
N_EXP = 64
TEMP = 0.1
N_TOK = 32768
BT = 4096
NCHUNK = 1
CH = N_TOK // NCHUNK

_info = plsc.get_sparse_core_info()
_NC, _NS, _L = _info.num_cores, _info.num_subcores, _info.num_lanes
_NW = _NC * _NS


def _logits_body(x_ref, wt_ref, b_ref, logits_ref, logits_t_ref):
    logits = (
        jnp.dot(x_ref[...], wt_ref[...], preferred_element_type=jnp.float32)
        + b_ref[...][None, :]) / TEMP
    logits_ref[...] = logits
    logits_t_ref[...] = logits.T


def _logits_body_acc(x_ref, wt_ref, b_ref, prev_ref, logits_ref, logits_t_ref):
    del prev_ref
    _logits_body(x_ref, wt_ref, b_ref, logits_ref, logits_t_ref)


def _make_sc_topk(n_tok):
    tok_w = n_tok // _NW
    grp = tok_w // _L
    unroll = min(4, grp)
    mesh = plsc.VectorSubcoreMesh(core_axis_name="c", subcore_axis_name="s")

    npiece = 1

    @functools.partial(
        pl.kernel,
        mesh=mesh,
        out_type=[
            jax.ShapeDtypeStruct((2, n_tok), jnp.float32),
            jax.ShapeDtypeStruct((2, n_tok), jnp.int32),
        ],
        scratch_types=[
            pltpu.VMEM((N_EXP, tok_w), jnp.float32),
            pltpu.VMEM((tok_w,), jnp.float32),
            pltpu.VMEM((tok_w,), jnp.float32),
            pltpu.VMEM((tok_w,), jnp.int32),
            pltpu.VMEM((tok_w,), jnp.int32),
        ] + [pltpu.SemaphoreType.DMA] * npiece,
    )
    def _sc_topk(logits_t_hbm, probs_hbm, idx_hbm, lt_v, p1_v, p2_v, i1_v, i2_v,
                 *sems):
        wid = lax.axis_index("s") * _NC + lax.axis_index("c")
        base = wid * tok_w
        piece = tok_w // npiece
        copies = [
            pltpu.async_copy(
                logits_t_hbm.at[:, pl.ds(base + h * piece, piece)],
                lt_v.at[:, pl.ds(h * piece, piece)],
                sems[h])
            for h in range(npiece)
        ]

        neg = jnp.full((_L,), -jnp.inf, jnp.float32)
        zero = jnp.zeros((_L,), jnp.int32)

        def super_group(sg, _):
            offs = [sg * (unroll * _L) + g * _L for g in range(unroll)]
            m1 = [neg] * unroll
            m2 = [neg] * unroll
            j1 = [zero] * unroll
            j2 = [zero] * unroll
            for e in range(N_EXP):
                ei = jnp.full((_L,), e, jnp.int32)
                for g in range(unroll):
                    v = lt_v[e, pl.ds(offs[g], _L)]
                    gt1 = v > m1[g]
                    lose = jnp.minimum(v, m1[g])
                    gt2 = lose > m2[g]
                    nj1 = jnp.where(gt1, ei, j1[g])
                    tj = jnp.where(gt1, j1[g], ei)
                    j2[g] = jnp.where(gt2, tj, j2[g])
                    m1[g] = jnp.maximum(v, m1[g])
                    m2[g] = jnp.maximum(lose, m2[g])
                    j1[g] = nj1
            for g in range(unroll):
                e2 = jnp.exp(m2[g] - m1[g])
                p1 = 1.0 / (1.0 + e2)
                p1_v[pl.ds(offs[g], _L)] = p1
                p2_v[pl.ds(offs[g], _L)] = e2 * p1
                i1_v[pl.ds(offs[g], _L)] = j1[g]
                i2_v[pl.ds(offs[g], _L)] = j2[g]
            return 0

        spg = grp // unroll // npiece
        for h in range(npiece):
            copies[h].wait()
            lax.fori_loop(h * spg, (h + 1) * spg, super_group, 0)

        pltpu.sync_copy(p1_v, probs_hbm.at[0, pl.ds(base, tok_w)])
        pltpu.sync_copy(p2_v, probs_hbm.at[1, pl.ds(base, tok_w)])
        pltpu.sync_copy(i1_v, idx_hbm.at[0, pl.ds(base, tok_w)])
        pltpu.sync_copy(i2_v, idx_hbm.at[1, pl.ds(base, tok_w)])

    return _sc_topk


_sc_topk_chunk = _make_sc_topk(CH)


@jax.jit
def kernel(x, W, b):
    wt = W.T
    nblk = CH // BT
    logits = None
    probs_parts, idx_parts = [], []
    for c in range(NCHUNK):
        c0 = c * nblk
        x_spec = pl.BlockSpec((BT, D_MODEL), lambda i, c0=c0: (c0 + i, 0))
        w_spec = pl.BlockSpec((D_MODEL, N_EXP), lambda i: (0, 0))
        b_spec = pl.BlockSpec((N_EXP,), lambda i: (0,))
        out_specs = [
            pl.BlockSpec((BT, N_EXP), lambda i, c0=c0: (c0 + i, 0)),
            pl.BlockSpec((N_EXP, BT), lambda i: (0, i)),
        ]
        out_shape = [
            jax.ShapeDtypeStruct((N_TOK, N_EXP), jnp.float32),
            jax.ShapeDtypeStruct((N_EXP, CH), jnp.float32),
        ]
        if c == 0:
            logits, lt_c = pl.pallas_call(
                _logits_body,
                grid=(nblk,),
                in_specs=[x_spec, w_spec, b_spec],
                out_specs=out_specs,
                out_shape=out_shape,
            )(x, wt, b)
        else:
            logits, lt_c = pl.pallas_call(
                _logits_body_acc,
                grid=(nblk,),
                in_specs=[x_spec, w_spec, b_spec,
                          pl.BlockSpec(memory_space=pl.ANY)],
                out_specs=out_specs,
                out_shape=out_shape,
                input_output_aliases={3: 0},
            )(x, wt, b, logits)
        probs_c, idx_c = _sc_topk_chunk(lt_c)
        probs_parts.append(probs_c)
        idx_parts.append(idx_c)
    probs_t = jnp.concatenate(probs_parts, axis=1)
    idx_t = jnp.concatenate(idx_parts, axis=1)
    return (logits, probs_t.T, idx_t.T)


# speedup vs baseline: 1.0532x; 1.0157x over previous
import functools

import jax
import jax.numpy as jnp
from jax import lax
from jax.experimental import pallas as pl
from jax.experimental.pallas import tpu as pltpu
from jax.experimental.pallas import tpu_sc as plsc

D_MODEL = 768
N_EXP = 64
HALF = N_EXP // 2
TEMP = 0.1
N_TOK = 32768
BT = 4096

_info = plsc.get_sparse_core_info()
_NC, _NS, _L = _info.num_cores, _info.num_subcores, _info.num_lanes
_NW = _NC * _NS
TOK_W = N_TOK // _NW
_GRP = TOK_W // _L
_UNROLL = 4


def _logits_body(x_ref, wt_ref, b_ref, logits_ref, pmax_ref, pminb_ref):
    logits = (
        jnp.dot(x_ref[...], wt_ref[...], preferred_element_type=jnp.float32)
        + b_ref[...][None, :]) / TEMP
    logits_ref[...] = logits
    lt = logits.T
    a = lt[:HALF, :]
    b2 = lt[HALF:, :]
    pmax = jnp.maximum(a, b2)
    pmin = jnp.minimum(a, b2)
    bit = (a >= b2).astype(jnp.int32)
    pmin_i = lax.bitcast_convert_type(pmin, jnp.int32)
    pmax_ref[...] = pmax
    pminb_ref[...] = lax.bitcast_convert_type((pmin_i & -2) | bit, jnp.float32)


_sc_mesh = plsc.VectorSubcoreMesh(core_axis_name="c", subcore_axis_name="s")


@functools.partial(
    pl.kernel,
    mesh=_sc_mesh,
    out_type=[
        jax.ShapeDtypeStruct((2, N_TOK), jnp.float32),
        jax.ShapeDtypeStruct((2, N_TOK), jnp.int32),
    ],
    scratch_types=[
        pltpu.VMEM((HALF, TOK_W), jnp.float32),
        pltpu.VMEM((HALF, TOK_W), jnp.float32),
        pltpu.VMEM((TOK_W,), jnp.float32),
        pltpu.VMEM((TOK_W,), jnp.float32),
        pltpu.VMEM((TOK_W,), jnp.int32),
        pltpu.VMEM((TOK_W,), jnp.int32),
    ],
)
def _sc_topk(pmax_hbm, pminb_hbm, probs_hbm, idx_hbm,
             pmax_v, pmb_v, p1_v, p2_v, i1_v, i2_v):
    wid = lax.axis_index("s") * _NC + lax.axis_index("c")
    base = wid * TOK_W
    pltpu.sync_copy(pmax_hbm.at[:, pl.ds(base, TOK_W)], pmax_v)
    pltpu.sync_copy(pminb_hbm.at[:, pl.ds(base, TOK_W)], pmb_v)

    neg = jnp.full((_L,), -jnp.inf, jnp.float32)
    zero = jnp.zeros((_L,), jnp.int32)
    zerof = jnp.zeros((_L,), jnp.float32)

    def super_group(sg, _):
        offs = [sg * (_UNROLL * _L) + g * _L for g in range(_UNROLL)]
        m1 = [neg] * _UNROLL
        m2 = [neg] * _UNROLL
        j1p = [zero] * _UNROLL
        j2p = [zero] * _UNROLL
        winm = [zerof] * _UNROLL
        renc = [zerof] * _UNROLL
        for p in range(HALF):
            pi = jnp.full((_L,), p, jnp.int32)
            for g in range(_UNROLL):
                v = pmax_v[p, pl.ds(offs[g], _L)]
                w = pmb_v[p, pl.ds(offs[g], _L)]
                gt1 = v > m1[g]
                lose = jnp.minimum(v, m1[g])
                gt2 = lose > m2[g]
                nj1 = jnp.where(gt1, pi, j1p[g])
                tjp = jnp.where(gt1, j1p[g], pi)
                twe = jnp.where(gt1, winm[g], w)
                j2p[g] = jnp.where(gt2, tjp, j2p[g])
                renc[g] = jnp.where(gt2, twe, renc[g])
                winm[g] = jnp.where(gt1, w, winm[g])
                m1[g] = jnp.maximum(v, m1[g])
                m2[g] = jnp.maximum(lose, m2[g])
                j1p[g] = nj1
        for g in range(_UNROLL):
            wi = lax.bitcast_convert_type(winm[g], jnp.int32)
            bit1 = wi & 1
            wval = lax.bitcast_convert_type(wi & -2, jnp.float32)
            ri = lax.bitcast_convert_type(renc[g], jnp.int32)
            bit2 = ri & 1
            j1 = j1p[g] + (1 - bit1) * HALF
            loser = j1p[g] + bit1 * HALF
            j2cand = j2p[g] + (1 - bit2) * HALF
            gtw = wval > m2[g]
            eqw = wval == m2[g]
            m2v = jnp.maximum(wval, m2[g])
            j2 = jnp.where(
                gtw, loser,
                jnp.where(eqw, jnp.minimum(loser, j2cand), j2cand))
            e2 = jnp.exp(m2v - m1[g])
            p1 = 1.0 / (1.0 + e2)
            p1_v[pl.ds(offs[g], _L)] = p1
            p2_v[pl.ds(offs[g], _L)] = e2 * p1
            i1_v[pl.ds(offs[g], _L)] = j1
            i2_v[pl.ds(offs[g], _L)] = j2
        return 0

    lax.fori_loop(0, _GRP // _UNROLL, super_group, 0)

    pltpu.sync_copy(p1_v, probs_hbm.at[0, pl.ds(base, TOK_W)])
    pltpu.sync_copy(p2_v, probs_hbm.at[1, pl.ds(base, TOK_W)])
    pltpu.sync_copy(i1_v, idx_hbm.at[0, pl.ds(base, TOK_W)])
    pltpu.sync_copy(i2_v, idx_hbm.at[1, pl.ds(base, TOK_W)])


@jax.jit
def kernel(x, W, b):
    n_tokens = x.shape[0]
    wt = W.T
    logits, pmax, pminb = pl.pallas_call(
        _logits_body,
        grid=(n_tokens // BT,),
        in_specs=[
            pl.BlockSpec((BT, D_MODEL), lambda i: (i, 0)),
            pl.BlockSpec((D_MODEL, N_EXP), lambda i: (0, 0)),
            pl.BlockSpec((N_EXP,), lambda i: (0,)),
        ],
        out_specs=[
            pl.BlockSpec((BT, N_EXP), lambda i: (i, 0)),
            pl.BlockSpec((HALF, BT), lambda i: (0, i)),
            pl.BlockSpec((HALF, BT), lambda i: (0, i)),
        ],
        out_shape=[
            jax.ShapeDtypeStruct((n_tokens, N_EXP), jnp.float32),
            jax.ShapeDtypeStruct((HALF, n_tokens), jnp.float32),
            jax.ShapeDtypeStruct((HALF, n_tokens), jnp.float32),
        ],
    )(x, wt, b)
    probs_t, idx_t = _sc_topk(pmax, pminb)
    return (logits, probs_t.T, idx_t.T)
